# Initial kernel scaffold; baseline (speedup 1.0000x reference)
import jax
import jax.numpy as jnp
from jax.experimental import pallas as pl


def kernel(x, mask, weight, bias):
    raise NotImplementedError("write your pallas kernel here")


# speedup vs baseline: 4.7283x; 4.7283x over previous
import functools

import jax
import jax.numpy as jnp
from jax import lax
from jax.experimental import pallas as pl
from jax.experimental.pallas import tpu as pltpu, tpu_sc as plsc

_NC, _NS, _L = 2, 16, 16
_NW = _NC * _NS
_B = 4096
_DH = 2048
_RPW = _B // _NW
_NG = _RPW // _L
_NB = 512
_NPASS = 3
_BIG = 3.0e38

_mesh = plsc.VectorSubcoreMesh(core_axis_name="c", subcore_axis_name="s")


@functools.partial(
    pl.kernel,
    mesh=_mesh,
    out_type=jax.ShapeDtypeStruct((_B,), jnp.float32),
    compiler_params=pltpu.CompilerParams(needs_layout_passes=False),
    scratch_types=[
        pltpu.VMEM((_L * _DH,), jnp.float32),
        pltpu.VMEM((_DH,), jnp.float32),
        pltpu.VMEM((2 * _DH,), jnp.float32),
        pltpu.VMEM((_L,), jnp.float32),
        pltpu.VMEM((_NB * _L,), jnp.float32),
        pltpu.VMEM((_RPW,), jnp.float32),
    ],
)
def _wos_sc(x_hbm, mask_hbm, w_hbm, bias_hbm, out_hbm,
            xv, mask_v, w_v, bias_v, hist, yv):
    wid = lax.axis_index("s") * _NC + lax.axis_index("c")
    pltpu.sync_copy(mask_hbm, mask_v)
    pltpu.sync_copy(w_hbm, w_v)
    pltpu.sync_copy(bias_hbm, bias_v)
    bias = bias_v[...]
    lane = lax.iota(jnp.int32, _L)
    gidx0 = lane * _DH
    zeros = jnp.zeros((_L,), jnp.float32)

    _CB = _DH // _L

    for g in range(_NG):
        base = (wid * _RPW + g * _L) * _DH
        pltpu.sync_copy(x_hbm.at[pl.ds(base, _L * _DH)], xv)

        def cols(j):
            mv = mask_v[pl.ds(j * _L, _L)]
            return [plsc.load_gather(xv, [gidx0 + (j * _L + k)]) + mv[k]
                    for k in range(_L)]

        def p_max(j, amax):
            for v in cols(j):
                amax = jnp.maximum(amax, jnp.abs(v))
            return amax
        amax = lax.fori_loop(0, _CB, p_max, zeros)
        lo = -amax - 1e-6
        hi = amax + 1e-6

        scale = None
        bhat = None
        for p in range(_NPASS):
            def p_zero(b, _):
                hist[pl.ds(b * _L, _L)] = zeros
                return 0
            lax.fori_loop(0, _NB, p_zero, 0)

            scale = _NB / (hi - lo)

            def p_hist(j, _, lo=lo, scale=scale):
                wp = w_v[pl.ds(j * _L, _L)]
                wn = w_v[pl.ds(_DH + j * _L, _L)]
                for k, v in enumerate(cols(j)):
                    b1 = jnp.clip((v - lo) * scale, 0.0, _NB - 1.0).astype(jnp.int32)
                    plsc.addupdate_scatter(
                        hist, [b1 * _L + lane], jnp.broadcast_to(wp[k], (_L,)))
                    b2 = jnp.clip((-v - lo) * scale, 0.0, _NB - 1.0).astype(jnp.int32)
                    plsc.addupdate_scatter(
                        hist, [b2 * _L + lane], jnp.broadcast_to(wn[k], (_L,)))
                return 0
            lax.fori_loop(0, _CB, p_hist, 0)

            def p_scan(i, carry):
                acc, bhat = carry
                b = _NB - 2 - i
                acc = acc + hist[pl.ds((b + 1) * _L, _L)]
                bhat = jnp.where(acc <= bias, b, bhat)
                return acc, bhat
            _, bhat = lax.fori_loop(
                0, _NB - 1, p_scan,
                (zeros, jnp.full((_L,), _NB - 1, jnp.int32)))
            if p < _NPASS - 1:
                bw = (hi - lo) * (1.0 / _NB)
                lo = lo + bhat.astype(jnp.float32) * bw
                hi = lo + bw

        def p_min(j, ymin, lo=lo, scale=scale, bhat=bhat):
            for v in cols(j):
                b1 = jnp.clip((v - lo) * scale, 0.0, float(_NB)).astype(jnp.int32)
                b2 = jnp.clip((-v - lo) * scale, 0.0, float(_NB)).astype(jnp.int32)
                y1 = jnp.where(b1 > bhat, v, _BIG)
                y2 = jnp.where(b2 > bhat, -v, _BIG)
                ymin = jnp.minimum(ymin, jnp.minimum(y1, y2))
            return ymin
        ymin = lax.fori_loop(0, _CB, p_min, jnp.full((_L,), _BIG, jnp.float32))
        yv[pl.ds(g * _L, _L)] = ymin

    pltpu.sync_copy(yv, out_hbm.at[pl.ds(wid * _RPW, _RPW)])


def kernel(x, mask, weight, bias):
    B = x.shape[0]
    xf = x.reshape(B * _DH).astype(jnp.float32)
    y = _wos_sc(xf, mask.reshape(-1), weight.reshape(-1),
                jnp.full((_L,), bias[0, 0], jnp.float32))
    return y.reshape(B, 1, 1, 1)
